# baseline (speedup 1.0000x reference)
import jax
import jax.numpy as jnp
from jax.experimental import pallas as pl
from jax.experimental.pallas import tpu as pltpu

_B, _C, _N, _E = 16, 7, 4096, 32004
_ND, _ED, _H = 7, 5, 64
_EBLK = 256
_EB = (_E + _EBLK - 1) // _EBLK
_EPAD = _EB * _EBLK
_NBLK = 1024
_NB = _N // _NBLK


def _gelu(x):
    return 0.5 * x * (1.0 + jax.lax.erf(x * 0.7071067811865476))


def _ln(z, g, b):
    mu = jnp.mean(z, axis=-1, keepdims=True)
    var = jnp.mean((z - mu) ** 2, axis=-1, keepdims=True)
    return (z - mu) / jnp.sqrt(var + 1e-5) * g + b


def _nrm(x):
    n = jnp.sqrt(jnp.sum(x * x, axis=-1, keepdims=True))
    return x / jnp.maximum(n, 1e-6)


def _stage1(x_ref, wn_ref, bn_ref, g1_ref, b1_ref, h_ref):
    z = jnp.dot(x_ref[0], wn_ref[...], preferred_element_type=jnp.float32)
    z = z + bn_ref[...]
    h_ref[0] = _gelu(_ln(z, g1_ref[...], b1_ref[...]))


def _stage2(ea_ref, srcT_ref, dstT_ref, h0_ref, pe_ref, ge_ref,
            we_ref, be_ref, g2_ref, b2_ref, wmt_ref, wmb_ref, bm_ref,
            agg_ref, nume_ref, dene_ref, accn, accd):
    j = pl.program_id(1)
    z = jnp.dot(ea_ref[0], we_ref[...], preferred_element_type=jnp.float32)
    z = z + be_ref[...]
    e = _gelu(_ln(z, g2_ref[...], b2_ref[...]))
    srcb = srcT_ref[0]
    oh = (jax.lax.broadcasted_iota(jnp.int32, (_EBLK, _N), 1)
          == srcb).astype(jnp.float32)
    u = jnp.dot(oh, h0_ref[0], preferred_element_type=jnp.float32)
    mpre = (jnp.dot(u, wmt_ref[...], preferred_element_type=jnp.float32)
            + jnp.dot(e, wmb_ref[...], preferred_element_type=jnp.float32)
            + bm_ref[...])
    m = _gelu(mpre)
    validc = (j * _EBLK + jax.lax.broadcasted_iota(jnp.int32, (_EBLK, 1), 0)) < _E
    m = jnp.where(validc, m, 0.0)
    dstb = dstT_ref[0]
    ohT = (jax.lax.broadcasted_iota(jnp.int32, (_N, _EBLK), 0)
           == dstb).astype(jnp.float32)
    contrib = jnp.dot(ohT, m, preferred_element_type=jnp.float32)

    @pl.when(j == 0)
    def _():
        agg_ref[0] = contrib

    @pl.when(j > 0)
    def _():
        agg_ref[0] = agg_ref[0] + contrib

    en = _nrm(e)
    pn = _nrm(pe_ref[...])
    sim = jnp.sum(pn * en[None], axis=-1)
    w = jax.nn.sigmoid(sim / 0.2) * jax.nn.sigmoid(ge_ref[...])
    validr = (j * _EBLK + jax.lax.broadcasted_iota(jnp.int32, (1, _EBLK), 1)) < _E
    w = jnp.where(validr, w, 0.0)
    pnum = jnp.sum((w * sim).reshape(_C, _EBLK // 128, 128), axis=1)
    pden = jnp.sum(w.reshape(_C, _EBLK // 128, 128), axis=1)

    @pl.when(j == 0)
    def _():
        accn[...] = pnum
        accd[...] = pden

    @pl.when(j > 0)
    def _():
        accn[...] = accn[...] + pnum
        accd[...] = accd[...] + pden

    @pl.when(j == _EB - 1)
    def _():
        nume_ref[0] = jnp.sum(accn[...], axis=1, keepdims=True)
        dene_ref[0] = jnp.sum(accd[...], axis=1, keepdims=True)


def _stage3(h0_ref, agg_ref, wu_ref, bu_ref, g3_ref, b3_ref, pn_ref, gn_ref,
            numn_ref, denn_ref, accn, accd):
    j = pl.program_id(1)
    upd = jnp.dot(agg_ref[0], wu_ref[...], preferred_element_type=jnp.float32)
    hf = _ln(h0_ref[0] + upd + bu_ref[...], g3_ref[...], b3_ref[...])
    hn = _nrm(hf)
    pn = _nrm(pn_ref[...])
    sim = jnp.sum(pn * hn[None], axis=-1)
    w = jax.nn.sigmoid(sim / 0.2) * jax.nn.sigmoid(gn_ref[...])
    pnum = jnp.sum((w * sim).reshape(_C, _NBLK // 128, 128), axis=1)
    pden = jnp.sum(w.reshape(_C, _NBLK // 128, 128), axis=1)

    @pl.when(j == 0)
    def _():
        accn[...] = pnum
        accd[...] = pden

    @pl.when(j > 0)
    def _():
        accn[...] = accn[...] + pnum
        accd[...] = accd[...] + pden

    @pl.when(j == _NB - 1)
    def _():
        numn_ref[0] = jnp.sum(accn[...], axis=1, keepdims=True)
        denn_ref[0] = jnp.sum(accd[...], axis=1, keepdims=True)


def kernel(x, edge_index, edge_attr, W_node, b_node, ln1_g, ln1_b,
           W_edge, b_edge, ln2_g, ln2_b, W_msg, b_msg, W_upd, b_upd,
           ln3_g, ln3_b, proto_n, proto_e, gate_n, gate_e):
    f32 = jnp.float32
    pad_e = _EPAD - _E
    ea_p = jnp.pad(edge_attr, ((0, 0), (0, pad_e), (0, 0)))
    pe_p = jnp.pad(proto_e, ((0, 0), (0, pad_e), (0, 0)))
    ge_p = jnp.pad(gate_e, ((0, 0), (0, pad_e)))
    srcT = jnp.pad(edge_index[0], (0, pad_e)).reshape(_EB, _EBLK, 1)
    dstT = jnp.pad(edge_index[1], (0, pad_e)).reshape(_EB, 1, _EBLK)
    bn = b_node.reshape(1, _H)
    g1 = ln1_g.reshape(1, _H)
    b1 = ln1_b.reshape(1, _H)
    be = b_edge.reshape(1, _H)
    g2 = ln2_g.reshape(1, _H)
    b2 = ln2_b.reshape(1, _H)
    bm = b_msg.reshape(1, _H)
    bu = b_upd.reshape(1, _H)
    g3 = ln3_g.reshape(1, _H)
    b3 = ln3_b.reshape(1, _H)
    wmt = W_msg[:_H]
    wmb = W_msg[_H:]

    h0 = pl.pallas_call(
        _stage1,
        grid=(_B, _NB),
        in_specs=[
            pl.BlockSpec((1, _NBLK, _ND), lambda b, j: (b, j, 0)),
            pl.BlockSpec((_ND, _H), lambda b, j: (0, 0)),
            pl.BlockSpec((1, _H), lambda b, j: (0, 0)),
            pl.BlockSpec((1, _H), lambda b, j: (0, 0)),
            pl.BlockSpec((1, _H), lambda b, j: (0, 0)),
        ],
        out_specs=pl.BlockSpec((1, _NBLK, _H), lambda b, j: (b, j, 0)),
        out_shape=jax.ShapeDtypeStruct((_B, _N, _H), f32),
    )(x, W_node, bn, g1, b1)

    agg, num_e, den_e = pl.pallas_call(
        _stage2,
        grid=(_B, _EB),
        in_specs=[
            pl.BlockSpec((1, _EBLK, _ED), lambda b, j: (b, j, 0)),
            pl.BlockSpec((1, _EBLK, 1), lambda b, j: (j, 0, 0)),
            pl.BlockSpec((1, 1, _EBLK), lambda b, j: (j, 0, 0)),
            pl.BlockSpec((1, _N, _H), lambda b, j: (b, 0, 0)),
            pl.BlockSpec((_C, _EBLK, _H), lambda b, j: (0, j, 0)),
            pl.BlockSpec((_C, _EBLK), lambda b, j: (0, j)),
            pl.BlockSpec((_ED, _H), lambda b, j: (0, 0)),
            pl.BlockSpec((1, _H), lambda b, j: (0, 0)),
            pl.BlockSpec((1, _H), lambda b, j: (0, 0)),
            pl.BlockSpec((1, _H), lambda b, j: (0, 0)),
            pl.BlockSpec((_H, _H), lambda b, j: (0, 0)),
            pl.BlockSpec((_H, _H), lambda b, j: (0, 0)),
            pl.BlockSpec((1, _H), lambda b, j: (0, 0)),
        ],
        out_specs=[
            pl.BlockSpec((1, _N, _H), lambda b, j: (b, 0, 0)),
            pl.BlockSpec((1, _C, 1), lambda b, j: (b, 0, 0)),
            pl.BlockSpec((1, _C, 1), lambda b, j: (b, 0, 0)),
        ],
        out_shape=[
            jax.ShapeDtypeStruct((_B, _N, _H), f32),
            jax.ShapeDtypeStruct((_B, _C, 1), f32),
            jax.ShapeDtypeStruct((_B, _C, 1), f32),
        ],
        scratch_shapes=[
            pltpu.VMEM((_C, 128), f32),
            pltpu.VMEM((_C, 128), f32),
        ],
    )(ea_p, srcT, dstT, h0, pe_p, ge_p, W_edge, be, g2, b2, wmt, wmb, bm)

    num_n, den_n = pl.pallas_call(
        _stage3,
        grid=(_B, _NB),
        in_specs=[
            pl.BlockSpec((1, _NBLK, _H), lambda b, j: (b, j, 0)),
            pl.BlockSpec((1, _NBLK, _H), lambda b, j: (b, j, 0)),
            pl.BlockSpec((_H, _H), lambda b, j: (0, 0)),
            pl.BlockSpec((1, _H), lambda b, j: (0, 0)),
            pl.BlockSpec((1, _H), lambda b, j: (0, 0)),
            pl.BlockSpec((1, _H), lambda b, j: (0, 0)),
            pl.BlockSpec((_C, _NBLK, _H), lambda b, j: (0, j, 0)),
            pl.BlockSpec((_C, _NBLK), lambda b, j: (0, j)),
        ],
        out_specs=[
            pl.BlockSpec((1, _C, 1), lambda b, j: (b, 0, 0)),
            pl.BlockSpec((1, _C, 1), lambda b, j: (b, 0, 0)),
        ],
        out_shape=[
            jax.ShapeDtypeStruct((_B, _C, 1), f32),
            jax.ShapeDtypeStruct((_B, _C, 1), f32),
        ],
        scratch_shapes=[
            pltpu.VMEM((_C, 128), f32),
            pltpu.VMEM((_C, 128), f32),
        ],
    )(h0, agg, W_upd, bu, g3, b3, proto_n, gate_n)

    ns = num_n[..., 0] / jnp.maximum(den_n[..., 0], 1e-6)
    es = num_e[..., 0] / jnp.maximum(den_e[..., 0], 1e-6)
    return ns + 0.5 * es


# speedup vs baseline: 2.8965x; 2.8965x over previous
import jax
import jax.numpy as jnp
from jax import lax
from jax.experimental import pallas as pl
from jax.experimental.pallas import tpu as pltpu
from jax.experimental.pallas import tpu_sc as plsc

_B, _C, _N, _E = 16, 7, 4096, 32004
_ND, _ED, _H = 7, 5, 64
_BP = _B // 2
_H2 = 2 * _H
_EPAD = 32768
_EBLK = 1024
_EB = _EPAD // _EBLK
_NBLK = 1024
_NB = _N // _NBLK

_EHW = _EPAD // 4
_GK = 4
_GG = _EHW // (128 * _GK)
_EPW = _EPAD // 16
_SK = 4
_SG = _EPW // (128 * _SK)
_NPT = _N // 16


def _gelu(x):
    return 0.5 * x * (1.0 + jax.lax.erf(x * 0.7071067811865476))


def _ln(z, g, b):
    mu = jnp.mean(z, axis=-1, keepdims=True)
    var = jnp.mean((z - mu) ** 2, axis=-1, keepdims=True)
    return (z - mu) / jnp.sqrt(var + 1e-5) * g + b


def _ln2(z, g, b):
    return jnp.concatenate(
        [_ln(z[:, :_H], g, b), _ln(z[:, _H:], g, b)], axis=-1)


def _nrm(x):
    n = jnp.sqrt(jnp.sum(x * x, axis=-1, keepdims=True))
    return x / jnp.maximum(n, 1e-6)


def _stage1(x_ref, wn_ref, bn_ref, g1_ref, b1_ref, h_ref):
    z = jnp.dot(x_ref[0], wn_ref[...], preferred_element_type=jnp.float32)
    z = z + bn_ref[...]
    h_ref[0] = _gelu(_ln2(z, g1_ref[...], b1_ref[...]))


def _sc_gather(h0_hbm, src_hbm, out_hbm, idx_v, rows_v, gsem, osem):
    c = lax.axis_index("c")
    s = lax.axis_index("s")
    w = s * 2 + c
    p = w // 4
    quarter = w % 4
    base_e = quarter * _EHW
    pltpu.sync_copy(src_hbm.at[pl.ds(base_e, _EHW)], idx_v)
    pn = p * _N

    def _addbase(i, _):
        idx_v[pl.ds(i * 16, 16)] = idx_v[pl.ds(i * 16, 16)] + pn
        return 0

    lax.fori_loop(0, _EHW // 16, _addbase, 0)

    def _group(g, _):
        hs = []
        for k in range(_GK):
            j = g * _GK + k
            hs.append(pltpu.async_copy(
                h0_hbm.at[idx_v.at[pl.ds(j * 128, 128)]], rows_v.at[k], gsem))
        for k in range(_GK):
            hs[k].wait()
        os = []
        for k in range(_GK):
            j = g * _GK + k
            os.append(pltpu.async_copy(
                rows_v.at[k],
                out_hbm.at[p, pl.ds(base_e + j * 128, 128), :], osem))
        for k in range(_GK):
            os[k].wait()
        return 0

    lax.fori_loop(0, _GG, _group, 0)


def _stage2(ea_ref, hs_ref, pe_ref, ge_ref,
            we_ref, be_ref, g2_ref, b2_ref, wmt_ref, wmb_ref, bm_ref,
            m_ref, nume_ref, dene_ref, an0, ad0, an1, ad1):
    j = pl.program_id(1)
    z = jnp.dot(ea_ref[0], we_ref[...], preferred_element_type=jnp.float32)
    z = z + be_ref[...]
    e = _gelu(_ln2(z, g2_ref[...], b2_ref[...]))
    mpre = (jnp.dot(hs_ref[0], wmt_ref[...], preferred_element_type=jnp.float32)
            + jnp.dot(e, wmb_ref[...], preferred_element_type=jnp.float32)
            + bm_ref[...])
    m = _gelu(mpre)
    validc = (j * _EBLK + jax.lax.broadcasted_iota(jnp.int32, (_EBLK, 1), 0)) < _E
    m_ref[0] = jnp.where(validc, m, 0.0)

    pnr = _nrm(pe_ref[...])
    ges = jax.nn.sigmoid(ge_ref[...])
    validr = (j * _EBLK + jax.lax.broadcasted_iota(jnp.int32, (1, _EBLK), 1)) < _E

    en0 = _nrm(e[:, :_H])
    en1 = _nrm(e[:, _H:])
    sim0 = jnp.sum(pnr * en0[None], axis=-1)
    sim1 = jnp.sum(pnr * en1[None], axis=-1)
    w0 = jnp.where(validr, jax.nn.sigmoid(sim0 / 0.2) * ges, 0.0)
    w1 = jnp.where(validr, jax.nn.sigmoid(sim1 / 0.2) * ges, 0.0)
    pn0 = jnp.sum((w0 * sim0).reshape(_C, _EBLK // 128, 128), axis=1)
    pd0 = jnp.sum(w0.reshape(_C, _EBLK // 128, 128), axis=1)
    pn1 = jnp.sum((w1 * sim1).reshape(_C, _EBLK // 128, 128), axis=1)
    pd1 = jnp.sum(w1.reshape(_C, _EBLK // 128, 128), axis=1)

    @pl.when(j == 0)
    def _():
        an0[...] = pn0
        ad0[...] = pd0
        an1[...] = pn1
        ad1[...] = pd1

    @pl.when(j > 0)
    def _():
        an0[...] = an0[...] + pn0
        ad0[...] = ad0[...] + pd0
        an1[...] = an1[...] + pn1
        ad1[...] = ad1[...] + pd1

    @pl.when(j == _EB - 1)
    def _():
        nume_ref[0, 0] = jnp.sum(an0[...], axis=1, keepdims=True)
        nume_ref[0, 1] = jnp.sum(an1[...], axis=1, keepdims=True)
        dene_ref[0, 0] = jnp.sum(ad0[...], axis=1, keepdims=True)
        dene_ref[0, 1] = jnp.sum(ad1[...], axis=1, keepdims=True)


def _sc_scatter(m_hbm, dst3_hbm, zer_hbm, agg_hbm,
                dst_v, rows_v, z_v, acc_sh, lsem):
    c = lax.axis_index("c")
    s = lax.axis_index("s")
    pltpu.sync_copy(dst3_hbm.at[s], dst_v)
    pltpu.sync_copy(zer_hbm, z_v)

    def _pair(k, _):
        p = c * (_BP // 2) + k
        pltpu.sync_copy(z_v, acc_sh.at[pl.ds(s * _NPT, 128)])
        pltpu.sync_copy(z_v, acc_sh.at[pl.ds(s * _NPT + 128, 128)])
        plsc.subcore_barrier()

        def _group(g, _):
            hs = []
            for t in range(_SK):
                cc = g * _SK + t
                hs.append(pltpu.async_copy(
                    m_hbm.at[p, pl.ds(s * _EPW + cc * 128, 128), :],
                    rows_v.at[t], lsem))
            for t in range(_SK):
                cc = g * _SK + t
                hs[t].wait()
                pltpu.sync_copy(rows_v.at[t], acc_sh.at[dst_v.at[cc]], add=True)
            return 0

        lax.fori_loop(0, _SG, _group, 0)
        plsc.subcore_barrier()
        pltpu.sync_copy(acc_sh.at[pl.ds(s * _NPT, _NPT)],
                        agg_hbm.at[p, pl.ds(s * _NPT, _NPT), :])
        plsc.subcore_barrier()
        return 0

    lax.fori_loop(0, _BP // 2, _pair, 0)


def _stage3(h0_ref, agg_ref, wu_ref, bu_ref, g3_ref, b3_ref, pn_ref, gn_ref,
            numn_ref, denn_ref, an0, ad0, an1, ad1):
    j = pl.program_id(1)
    upd = jnp.dot(agg_ref[0], wu_ref[...], preferred_element_type=jnp.float32)
    hf = _ln2(h0_ref[0] + upd + bu_ref[...], g3_ref[...], b3_ref[...])
    pnr = _nrm(pn_ref[...])
    gns = jax.nn.sigmoid(gn_ref[...])

    hn0 = _nrm(hf[:, :_H])
    hn1 = _nrm(hf[:, _H:])
    sim0 = jnp.sum(pnr * hn0[None], axis=-1)
    sim1 = jnp.sum(pnr * hn1[None], axis=-1)
    w0 = jax.nn.sigmoid(sim0 / 0.2) * gns
    w1 = jax.nn.sigmoid(sim1 / 0.2) * gns
    pn0 = jnp.sum((w0 * sim0).reshape(_C, _NBLK // 128, 128), axis=1)
    pd0 = jnp.sum(w0.reshape(_C, _NBLK // 128, 128), axis=1)
    pn1 = jnp.sum((w1 * sim1).reshape(_C, _NBLK // 128, 128), axis=1)
    pd1 = jnp.sum(w1.reshape(_C, _NBLK // 128, 128), axis=1)

    @pl.when(j == 0)
    def _():
        an0[...] = pn0
        ad0[...] = pd0
        an1[...] = pn1
        ad1[...] = pd1

    @pl.when(j > 0)
    def _():
        an0[...] = an0[...] + pn0
        ad0[...] = ad0[...] + pd0
        an1[...] = an1[...] + pn1
        ad1[...] = ad1[...] + pd1

    @pl.when(j == _NB - 1)
    def _():
        numn_ref[0, 0] = jnp.sum(an0[...], axis=1, keepdims=True)
        numn_ref[0, 1] = jnp.sum(an1[...], axis=1, keepdims=True)
        denn_ref[0, 0] = jnp.sum(ad0[...], axis=1, keepdims=True)
        denn_ref[0, 1] = jnp.sum(ad1[...], axis=1, keepdims=True)


def _blkdiag(w):
    k, n = w.shape
    z = jnp.zeros((2 * k, 2 * n), w.dtype)
    return z.at[:k, :n].set(w).at[k:, n:].set(w)


def kernel(x, edge_index, edge_attr, W_node, b_node, ln1_g, ln1_b,
           W_edge, b_edge, ln2_g, ln2_b, W_msg, b_msg, W_upd, b_upd,
           ln3_g, ln3_b, proto_n, proto_e, gate_n, gate_e):
    f32 = jnp.float32
    pad_e = _EPAD - _E
    xp = jnp.concatenate([x[0::2], x[1::2]], axis=-1)
    ea = jnp.pad(edge_attr, ((0, 0), (0, pad_e), (0, 0)))
    ea_p = jnp.concatenate([ea[0::2], ea[1::2]], axis=-1)
    pe_p = jnp.pad(proto_e, ((0, 0), (0, pad_e), (0, 0)))
    ge_p = jnp.pad(gate_e, ((0, 0), (0, pad_e)))
    src_p = jnp.pad(edge_index[0], (0, pad_e))
    dst3 = jnp.pad(edge_index[1], (0, pad_e)).reshape(16, _EPW // 128, 128)
    zer = jnp.zeros((128, _H2), f32)
    wn2 = _blkdiag(W_node)
    we2 = _blkdiag(W_edge)
    wmt2 = _blkdiag(W_msg[:_H])
    wmb2 = _blkdiag(W_msg[_H:])
    wu2 = _blkdiag(W_upd)
    bn2 = jnp.tile(b_node, 2).reshape(1, _H2)
    be2 = jnp.tile(b_edge, 2).reshape(1, _H2)
    bm2 = jnp.tile(b_msg, 2).reshape(1, _H2)
    bu2 = jnp.tile(b_upd, 2).reshape(1, _H2)
    g1 = ln1_g.reshape(1, _H)
    b1 = ln1_b.reshape(1, _H)
    g2 = ln2_g.reshape(1, _H)
    b2 = ln2_b.reshape(1, _H)
    g3 = ln3_g.reshape(1, _H)
    b3 = ln3_b.reshape(1, _H)

    h0p = pl.pallas_call(
        _stage1,
        grid=(_BP, _NB),
        in_specs=[
            pl.BlockSpec((1, _NBLK, 2 * _ND), lambda b, j: (b, j, 0)),
            pl.BlockSpec((2 * _ND, _H2), lambda b, j: (0, 0)),
            pl.BlockSpec((1, _H2), lambda b, j: (0, 0)),
            pl.BlockSpec((1, _H), lambda b, j: (0, 0)),
            pl.BlockSpec((1, _H), lambda b, j: (0, 0)),
        ],
        out_specs=pl.BlockSpec((1, _NBLK, _H2), lambda b, j: (b, j, 0)),
        out_shape=jax.ShapeDtypeStruct((_BP, _N, _H2), f32),
    )(xp, wn2, bn2, g1, b1)

    mesh = plsc.VectorSubcoreMesh(core_axis_name="c", subcore_axis_name="s")
    h0f = h0p.reshape(_BP * _N, _H2)
    h_src = pl.kernel(
        _sc_gather,
        mesh=mesh,
        out_type=jax.ShapeDtypeStruct((_BP, _EPAD, _H2), f32),
        scratch_types=[
            pltpu.VMEM((_EHW,), jnp.int32),
            pltpu.VMEM((_GK, 128, _H2), f32),
            pltpu.SemaphoreType.DMA,
            pltpu.SemaphoreType.DMA,
        ],
    )(h0f, src_p)

    m, num_e, den_e = pl.pallas_call(
        _stage2,
        grid=(_BP, _EB),
        in_specs=[
            pl.BlockSpec((1, _EBLK, 2 * _ED), lambda b, j: (b, j, 0)),
            pl.BlockSpec((1, _EBLK, _H2), lambda b, j: (b, j, 0)),
            pl.BlockSpec((_C, _EBLK, _H), lambda b, j: (0, j, 0)),
            pl.BlockSpec((_C, _EBLK), lambda b, j: (0, j)),
            pl.BlockSpec((2 * _ED, _H2), lambda b, j: (0, 0)),
            pl.BlockSpec((1, _H2), lambda b, j: (0, 0)),
            pl.BlockSpec((1, _H), lambda b, j: (0, 0)),
            pl.BlockSpec((1, _H), lambda b, j: (0, 0)),
            pl.BlockSpec((_H2, _H2), lambda b, j: (0, 0)),
            pl.BlockSpec((_H2, _H2), lambda b, j: (0, 0)),
            pl.BlockSpec((1, _H2), lambda b, j: (0, 0)),
        ],
        out_specs=[
            pl.BlockSpec((1, _EBLK, _H2), lambda b, j: (b, j, 0)),
            pl.BlockSpec((1, 2, _C, 1), lambda b, j: (b, 0, 0, 0)),
            pl.BlockSpec((1, 2, _C, 1), lambda b, j: (b, 0, 0, 0)),
        ],
        out_shape=[
            jax.ShapeDtypeStruct((_BP, _EPAD, _H2), f32),
            jax.ShapeDtypeStruct((_BP, 2, _C, 1), f32),
            jax.ShapeDtypeStruct((_BP, 2, _C, 1), f32),
        ],
        scratch_shapes=[
            pltpu.VMEM((_C, 128), f32),
            pltpu.VMEM((_C, 128), f32),
            pltpu.VMEM((_C, 128), f32),
            pltpu.VMEM((_C, 128), f32),
        ],
    )(ea_p, h_src, pe_p, ge_p, we2, be2, g2, b2, wmt2, wmb2, bm2)

    agg = pl.kernel(
        _sc_scatter,
        mesh=mesh,
        out_type=jax.ShapeDtypeStruct((_BP, _N, _H2), f32),
        scratch_types=[
            pltpu.VMEM((_EPW // 128, 128), jnp.int32),
            pltpu.VMEM((_SK, 128, _H2), f32),
            pltpu.VMEM((128, _H2), f32),
            pltpu.VMEM_SHARED((_N, _H2), f32),
            pltpu.SemaphoreType.DMA,
        ],
    )(m, dst3, zer)

    num_n, den_n = pl.pallas_call(
        _stage3,
        grid=(_BP, _NB),
        in_specs=[
            pl.BlockSpec((1, _NBLK, _H2), lambda b, j: (b, j, 0)),
            pl.BlockSpec((1, _NBLK, _H2), lambda b, j: (b, j, 0)),
            pl.BlockSpec((_H2, _H2), lambda b, j: (0, 0)),
            pl.BlockSpec((1, _H2), lambda b, j: (0, 0)),
            pl.BlockSpec((1, _H), lambda b, j: (0, 0)),
            pl.BlockSpec((1, _H), lambda b, j: (0, 0)),
            pl.BlockSpec((_C, _NBLK, _H), lambda b, j: (0, j, 0)),
            pl.BlockSpec((_C, _NBLK), lambda b, j: (0, j)),
        ],
        out_specs=[
            pl.BlockSpec((1, 2, _C, 1), lambda b, j: (b, 0, 0, 0)),
            pl.BlockSpec((1, 2, _C, 1), lambda b, j: (b, 0, 0, 0)),
        ],
        out_shape=[
            jax.ShapeDtypeStruct((_BP, 2, _C, 1), f32),
            jax.ShapeDtypeStruct((_BP, 2, _C, 1), f32),
        ],
        scratch_shapes=[
            pltpu.VMEM((_C, 128), f32),
            pltpu.VMEM((_C, 128), f32),
            pltpu.VMEM((_C, 128), f32),
            pltpu.VMEM((_C, 128), f32),
        ],
    )(h0p, agg, wu2, bu2, g3, b3, proto_n, gate_n)

    ns = num_n[..., 0].reshape(_B, _C) / jnp.maximum(
        den_n[..., 0].reshape(_B, _C), 1e-6)
    es = num_e[..., 0].reshape(_B, _C) / jnp.maximum(
        den_e[..., 0].reshape(_B, _C), 1e-6)
    return ns + 0.5 * es
